# baseline (speedup 1.0000x reference)
import jax
import jax.numpy as jnp
from jax.experimental import pallas as pl
from jax.experimental.pallas import tpu as pltpu

_N1 = 2048
_N2 = 2048
_D = 16


def _acosh(v):
    return jnp.log(v + jnp.sqrt(v * v - 1.0))


def _hausdorff_kernel(x_ref, y_ref, out_ref):
    xt = x_ref[...].T
    yt = y_ref[...].T
    xn = jnp.sum(xt * xt, axis=0, keepdims=True)
    yn = jnp.sum(yt * yt, axis=0, keepdims=True)
    c = 2.0 / (1.0 - xn)
    b = 1.0 / (1.0 - yn)
    axt = jnp.concatenate([xt * (-2.0 * c), xn * c, c], axis=0)
    ayt = jnp.concatenate([yt * b, b, yn * b], axis=0)
    m = jax.lax.dot_general(
        axt, ayt, (((0,), (0,)), ((), ())),
        preferred_element_type=jnp.float32)
    rmin = 1.0 + jnp.min(m, axis=1, keepdims=True)
    cmin = 1.0 + jnp.min(m, axis=0, keepdims=True)
    total = jnp.sum(_acosh(rmin)) / _N1 + jnp.sum(_acosh(cmin)) / _N2
    out_ref[...] = jnp.reshape(total, (1, 1))


def kernel(set1, set2):
    out = pl.pallas_call(
        _hausdorff_kernel,
        out_shape=jax.ShapeDtypeStruct((1, 1), jnp.float32),
        in_specs=[
            pl.BlockSpec(memory_space=pltpu.VMEM),
            pl.BlockSpec(memory_space=pltpu.VMEM),
        ],
        out_specs=pl.BlockSpec(memory_space=pltpu.VMEM),
    )(set1, set2)
    return out[0, 0]
